# baseline (speedup 1.0000x reference)
import functools

import jax
import jax.numpy as jnp
from jax import lax
from jax.experimental import pallas as pl
from jax.experimental.pallas import tpu as pltpu
from jax.experimental.pallas import tpu_sc as plsc

N = 10000
E = 320000
NP = 10240
HEADS1 = 8
MSG1 = 128
ROW1 = 136
MSG2 = 64
ROW2 = 72
NW = 32
EPW = E // NW
RPT = NP // 16


def _mm_kernel(x_ref, w_ref, o1_ref, o2_ref):
    t = jnp.dot(x_ref[...], w_ref[...], preferred_element_type=jnp.float32)
    o1_ref[...] = t[:, :ROW1]
    o2_ref[...] = t[:, ROW1:152]


def _tables1(x, w, bm=1000):
    m = x.shape[0]
    k = x.shape[1]
    return pl.pallas_call(
        _mm_kernel,
        grid=(m // bm,),
        in_specs=[pl.BlockSpec((bm, k), lambda i: (i, 0)),
                  pl.BlockSpec((k, 152), lambda i: (0, 0))],
        out_specs=[pl.BlockSpec((bm, ROW1), lambda i: (i, 0)),
                   pl.BlockSpec((bm, 16), lambda i: (i, 0))],
        out_shape=[jax.ShapeDtypeStruct((m, ROW1), jnp.float32),
                   jax.ShapeDtypeStruct((m, 16), jnp.float32)],
    )(x, w)


def _fin1_kernel(a0_ref, a1_ref, r_ref, b_ref, w_ref, o1_ref, o2_ref):
    acc = a0_ref[...] + a1_ref[...]
    msg = acc[:, :MSG1]
    s = acc[:, MSG1:MSG1 + HEADS1]
    s_exp = jnp.dot(s, r_ref[...], preferred_element_type=jnp.float32)
    h = msg / (s_exp + 1e-16) + b_ref[...]
    h = jnp.where(h > 0, h, jnp.exp(h) - 1.0)
    t = jnp.dot(h, w_ref[...], preferred_element_type=jnp.float32)
    o1_ref[...] = t[:, :ROW2]
    o2_ref[...] = t[:, ROW2:88]


def _finalize1_matmul2(a0, a1, rmat, b1, wbig2, bm=1024):
    return pl.pallas_call(
        _fin1_kernel,
        grid=(NP // bm,),
        in_specs=[pl.BlockSpec((bm, ROW1), lambda i: (i, 0)),
                  pl.BlockSpec((bm, ROW1), lambda i: (i, 0)),
                  pl.BlockSpec((HEADS1, MSG1), lambda i: (0, 0)),
                  pl.BlockSpec((1, MSG1), lambda i: (0, 0)),
                  pl.BlockSpec((MSG1, 88), lambda i: (0, 0))],
        out_specs=[pl.BlockSpec((bm, ROW2), lambda i: (i, 0)),
                   pl.BlockSpec((bm, 16), lambda i: (i, 0))],
        out_shape=[jax.ShapeDtypeStruct((NP, ROW2), jnp.float32),
                   jax.ShapeDtypeStruct((NP, 16), jnp.float32)],
    )(a0, a1, rmat, b1, wbig2)


def _fin2_kernel(a0_ref, a1_ref, b_ref, o_ref):
    acc = a0_ref[...] + a1_ref[...]
    msg = acc[:, :MSG2]
    s = acc[:, MSG2:MSG2 + 1]
    o_ref[...] = msg / (s + 1e-16) + b_ref[...]


def _finalize2(a0, a1, b2, bm=1024):
    return pl.pallas_call(
        _fin2_kernel,
        grid=(NP // bm,),
        in_specs=[pl.BlockSpec((bm, ROW2), lambda i: (i, 0)),
                  pl.BlockSpec((bm, ROW2), lambda i: (i, 0)),
                  pl.BlockSpec((1, MSG2), lambda i: (0, 0))],
        out_specs=pl.BlockSpec((bm, MSG2), lambda i: (i, 0)),
        out_shape=jax.ShapeDtypeStruct((NP, MSG2), jnp.float32),
    )(a0, a1, b2)


def _edge_pass(row_w, msg_w, heads, k, srctab, dsttab, eidx, zrows):
    cw = msg_w // heads
    nch = EPW // k
    mesh = plsc.VectorSubcoreMesh(core_axis_name="c", subcore_axis_name="s")

    scratch = [
        pltpu.VMEM((8, k), jnp.int32),
        pltpu.VMEM((8, k), jnp.int32),
        pltpu.VMEM((4, k, row_w), jnp.float32),
        pltpu.VMEM((4, k, 16), jnp.float32),
        pltpu.VMEM((2, k, row_w), jnp.float32),
        pltpu.VMEM_SHARED((NP, row_w), jnp.float32),
    ] + [pltpu.SemaphoreType.DMA] * 14

    @functools.partial(
        pl.kernel,
        mesh=mesh,
        compiler_params=pltpu.CompilerParams(use_tc_tiling_on_sc=False),
        out_type=jax.ShapeDtypeStruct((2, NP, row_w), jnp.float32),
        scratch_types=scratch,
    )
    def edge_kernel(srctab_hbm, dsttab_hbm, eidx_hbm, z_hbm,
                    out_hbm, sidx, didx, rbuf, dbuf, obuf, acc,
                    sg0, sg1, sg2, sg3, ss0, ss1,
                    si0, si1, si2, si3, si4, si5, si6, si7):
        cid = lax.axis_index("c")
        sid = lax.axis_index("s")
        wid = cid * 16 + sid
        sg = (sg0, sg1, sg2, sg3)
        ss = (ss0, ss1)
        si = (si0, si1, si2, si3, si4, si5, si6, si7)

        pltpu.sync_copy(z_hbm, acc.at[pl.ds(sid * RPT, RPT)])
        plsc.subcore_barrier()

        ebase = wid * EPW

        def issue_idx(c, q):
            pltpu.async_copy(eidx_hbm.at[0, pl.ds(ebase + c * k, k)],
                             sidx.at[q], si[q])
            pltpu.async_copy(eidx_hbm.at[1, pl.ds(ebase + c * k, k)],
                             didx.at[q], si[q])

        def wait_idx(c, q):
            pltpu.make_async_copy(eidx_hbm.at[0, pl.ds(ebase + c * k, k)],
                                  sidx.at[q], si[q]).wait()
            pltpu.make_async_copy(eidx_hbm.at[1, pl.ds(ebase + c * k, k)],
                                  didx.at[q], si[q]).wait()

        def issue_gather(q, g):
            pltpu.async_copy(srctab_hbm.at[sidx.at[q]], rbuf.at[g], sg[g])
            pltpu.async_copy(dsttab_hbm.at[didx.at[q]], dbuf.at[g], sg[g])

        def wait_gather(q, g):
            pltpu.make_async_copy(srctab_hbm.at[sidx.at[q]], rbuf.at[g],
                                  sg[g]).wait()
            pltpu.make_async_copy(dsttab_hbm.at[didx.at[q]], dbuf.at[g],
                                  sg[g]).wait()

        def wait_scatter(q, b):
            pltpu.make_async_copy(obuf.at[b], acc.at[didx.at[q]],
                                  ss[b]).wait()

        def compute_scatter(q, g, b):
            @plsc.parallel_loop(0, k, unroll=2)
            def ebody(i):
                ev = rbuf[g, i, pl.ds(msg_w - 8, 16)]
                dv = dbuf[g, i, pl.ds(0, 16)]
                e = ev + dv
                e = jnp.where(e >= 0, e, 0.2 * e)
                p = jnp.exp(e)
                obuf[b, i, pl.ds(msg_w - 8, 16)] = p
                for hd in range(heads):
                    ph = p[8 + hd]
                    for qq in range(cw // 16):
                        sl = hd * cw + qq * 16
                        obuf[b, i, pl.ds(sl, 16)] = (
                            rbuf[g, i, pl.ds(sl, 16)] * ph)
            pltpu.async_copy(obuf.at[b], acc.at[didx.at[q]], ss[b],
                             add=True)

        def step(c, u):
            wait_gather(u % 8, u % 4)

            @pl.when(c >= 2)
            def _():
                wait_scatter((u + 6) % 8, u % 2)

            wait_idx(c + 3, (u + 3) % 8)
            issue_gather((u + 3) % 8, (u + 3) % 4)
            issue_idx(c + 4, (u + 4) % 8)
            compute_scatter(u % 8, u % 4, u % 2)

        issue_idx(0, 0)
        issue_idx(1, 1)
        issue_idx(2, 2)
        issue_idx(3, 3)
        wait_idx(0, 0)
        issue_gather(0, 0)
        wait_idx(1, 1)
        issue_gather(1, 1)
        wait_idx(2, 2)
        issue_gather(2, 2)

        T = (nch - 4) // 8
        def outer(t, carry):
            c0 = t * 8
            for u in range(8):
                step(c0 + u, u)
            return carry
        lax.fori_loop(0, T, outer, 0)

        for c in range(8 * T, nch):
            wait_gather(c % 8, c % 4)
            wait_scatter((c - 2) % 8, c % 2)
            if c + 3 < nch:
                wait_idx(c + 3, (c + 3) % 8)
                issue_gather((c + 3) % 8, (c + 3) % 4)
            if c + 4 < nch:
                issue_idx(c + 4, (c + 4) % 8)
            compute_scatter(c % 8, c % 4, c % 2)
        wait_scatter((nch - 2) % 8, (nch - 2) % 2)
        wait_scatter((nch - 1) % 8, (nch - 1) % 2)

        plsc.subcore_barrier()

        pltpu.sync_copy(acc.at[pl.ds(sid * RPT, RPT)],
                        out_hbm.at[cid, pl.ds(sid * RPT, RPT)])

    return edge_kernel(srctab, dsttab, eidx, zrows)


@jax.jit
def kernel(x, edge_index, W1, a_src1, a_dst1, b1, W2, a_src2, a_dst2, b2):
    eidx = edge_index.astype(jnp.int32)

    w1h = W1.reshape(x.shape[1], HEADS1, 16)
    wsrc1 = jnp.einsum('ihc,hc->ih', w1h, a_src1)
    wdst1 = jnp.einsum('ihc,hc->ih', w1h, a_dst1)
    z8 = jnp.zeros((x.shape[1], HEADS1), jnp.float32)
    wbig1 = jnp.concatenate([W1, wsrc1, z8, wdst1], axis=1)
    eye8 = jnp.eye(HEADS1, dtype=jnp.float32)

    z7 = jnp.zeros((MSG1, 7), jnp.float32)
    z8b = jnp.zeros((MSG1, 8), jnp.float32)
    wbig2 = jnp.concatenate(
        [W2, (W2 @ a_src2[0])[:, None], z7, z8b, (W2 @ a_dst2[0])[:, None],
         z7], axis=1)
    rmat = jnp.repeat(eye8, 16, axis=1)

    srctab1, dsttab1 = _tables1(x, wbig1)
    z1 = jnp.zeros((RPT, ROW1), jnp.float32)
    accp1 = _edge_pass(ROW1, MSG1, HEADS1, 40, srctab1, dsttab1, eidx, z1)

    srctab2, dsttab2 = _finalize1_matmul2(accp1[0], accp1[1], rmat,
                                          b1.reshape(1, MSG1), wbig2)
    z2 = jnp.zeros((RPT, ROW2), jnp.float32)
    accp2 = _edge_pass(ROW2, MSG2, 1, 80, srctab2, dsttab2, eidx, z2)

    out = _finalize2(accp2[0], accp2[1], b2.reshape(1, MSG2))
    return out[:N]
